# Initial kernel scaffold; baseline (speedup 1.0000x reference)
import jax
import jax.numpy as jnp
from jax.experimental import pallas as pl


def kernel(inputs, table):
    raise NotImplementedError("write your pallas kernel here")


# speedup vs baseline: 2.2340x; 2.2340x over previous
import functools
import math

import jax
import jax.numpy as jnp
from jax.experimental import pallas as pl
from jax.experimental.pallas import tpu as pltpu
from jax.experimental.pallas import tpu_sc as plsc

_D = 128
_B = 4096 * 50
_W = 128
_SCALE = math.sqrt(float(_D))


def _scale_block(t_ref, o_ref):
    o_ref[...] = t_ref[...] * _SCALE


def _scaled_table(table):
    rows, d = table.shape
    blk = 1000
    return pl.pallas_call(
        _scale_block,
        out_shape=jax.ShapeDtypeStruct(table.shape, table.dtype),
        grid=(rows // blk,),
        in_specs=[pl.BlockSpec((blk, d), lambda i: (i, 0))],
        out_specs=pl.BlockSpec((blk, d), lambda i: (i, 0)),
    )(table)


def _sc_gather(table_scaled, idx_flat):
    mesh = plsc.VectorSubcoreMesh(core_axis_name="c", subcore_axis_name="s")

    @functools.partial(
        pl.kernel,
        out_type=jax.ShapeDtypeStruct((_B, _D), jnp.float32),
        mesh=mesh,
    )
    def k(t_hbm, i_hbm, o_hbm):
        def body(i_vmem, o_vmem):
            pltpu.sync_copy(t_hbm.at[i_vmem.at[0]], o_vmem)

        pltpu.emit_pipeline(
            body,
            grid=(_B // _W,),
            in_specs=[pl.BlockSpec((1, _W), index_map=lambda i: (0, i))],
            out_specs=[pl.BlockSpec((_W, _D), index_map=lambda i: (i, 0))],
            core_axis_name=("c", "s"),
            dimension_semantics=(pltpu.PARALLEL,),
        )(i_hbm, o_hbm)

    return k(table_scaled, idx_flat)


def kernel(inputs, table):
    idx = inputs.reshape(1, _B).astype(jnp.int32)
    ts = _scaled_table(table)
    out = _sc_gather(ts, idx)
    return out.reshape(inputs.shape[0], inputs.shape[1], _D)


# speedup vs baseline: 2.8364x; 1.2697x over previous
import functools
import math

import jax
import jax.numpy as jnp
from jax.experimental import pallas as pl
from jax.experimental.pallas import tpu as pltpu
from jax.experimental.pallas import tpu_sc as plsc

_D = 128
_B = 4096 * 50
_W = 128
_SCALE = math.sqrt(float(_D))


def _scale_block(t_ref, o_ref):
    o_ref[...] = t_ref[...] * _SCALE


def _scaled_table(table):
    rows, d = table.shape
    blk = 1000
    return pl.pallas_call(
        _scale_block,
        out_shape=jax.ShapeDtypeStruct(table.shape, table.dtype),
        grid=(rows // blk,),
        in_specs=[pl.BlockSpec((blk, d), lambda i: (i, 0))],
        out_specs=pl.BlockSpec((blk, d), lambda i: (i, 0)),
    )(table)


def _sc_gather(table_scaled, idx3, batch, seq):
    mesh = plsc.VectorSubcoreMesh(core_axis_name="c", subcore_axis_name="s")

    @functools.partial(
        pl.kernel,
        out_type=jax.ShapeDtypeStruct((batch, seq, _D), jnp.float32),
        mesh=mesh,
    )
    def k(t_hbm, i_hbm, o_hbm):
        def body(i_vmem, o_vmem):
            for j in range(_BW):
                pltpu.sync_copy(t_hbm.at[i_vmem.at[j, 0]], o_vmem.at[j])

        pltpu.emit_pipeline(
            body,
            grid=(batch // _BW,),
            in_specs=[pl.BlockSpec((_BW, 1, seq), index_map=lambda i: (i, 0, 0))],
            out_specs=[pl.BlockSpec((_BW, seq, _D), index_map=lambda i: (i, 0, 0))],
            core_axis_name=("c", "s"),
            dimension_semantics=(pltpu.PARALLEL,),
        )(i_hbm, o_hbm)

    return k(table_scaled, idx3)


_BW = 4


def kernel(inputs, table):
    batch, seq = inputs.shape
    idx3 = inputs.reshape(batch, 1, seq)
    ts = _scaled_table(table)
    return _sc_gather(ts, idx3, batch, seq)


# speedup vs baseline: 3.6254x; 1.2781x over previous
import functools
import math

import jax
import jax.numpy as jnp
from jax.experimental import pallas as pl
from jax.experimental.pallas import tpu as pltpu
from jax.experimental.pallas import tpu_sc as plsc

_D = 128
_B = 4096 * 50
_W = 128
_SCALE = math.sqrt(float(_D))


def _scale_block(t_ref, o_ref):
    o_ref[...] = t_ref[...] * _SCALE


def _scaled_table(table):
    rows, d = table.shape
    blk = 1000
    return pl.pallas_call(
        _scale_block,
        out_shape=jax.ShapeDtypeStruct(table.shape, table.dtype),
        grid=(rows // blk,),
        in_specs=[pl.BlockSpec((blk, d), lambda i: (i, 0))],
        out_specs=pl.BlockSpec((blk, d), lambda i: (i, 0)),
    )(table)


_BW = 8


def _sc_gather(table_scaled, idx, batch, seq):
    mesh = plsc.VectorSubcoreMesh(core_axis_name="c", subcore_axis_name="s")

    @functools.partial(
        pl.kernel,
        out_type=jax.ShapeDtypeStruct((batch, seq, _D), jnp.float32),
        mesh=mesh,
        scratch_types=[pltpu.SemaphoreType.DMA],
    )
    def k(t_hbm, i_hbm, o_hbm, sem):
        def body(i_vmem, o_vmem):
            copies = [
                pltpu.async_copy(t_hbm.at[i_vmem.at[j]], o_vmem.at[j], sem)
                for j in range(_BW)
            ]
            for c in copies:
                c.wait()

        pltpu.emit_pipeline(
            body,
            grid=(batch // _BW,),
            in_specs=[pl.BlockSpec((_BW, seq), index_map=lambda i: (i, 0))],
            out_specs=[pl.BlockSpec((_BW, seq, _D), index_map=lambda i: (i, 0, 0))],
            core_axis_name=("c", "s"),
            dimension_semantics=(pltpu.PARALLEL,),
        )(i_hbm, o_hbm)

    return k(table_scaled, idx)


def kernel(inputs, table):
    batch, seq = inputs.shape
    ts = _scaled_table(table)
    return _sc_gather(ts, inputs, batch, seq)
